# baseline (speedup 1.0000x reference)
import functools

import jax
import jax.numpy as jnp
from jax import lax
from jax.experimental import pallas as pl
from jax.experimental.pallas import tpu as pltpu
from jax.experimental.pallas import tpu_sc as plsc

SEQ_LEN = 8192
DIM = 1024
NUM_CORES = 2
NUM_SUBCORES = 16
NUM_WORKERS = NUM_CORES * NUM_SUBCORES

SC_ROWS = 1024
ROWS_PER_WORKER = SC_ROWS // NUM_WORKERS
CHUNK = 16
NCH = ROWS_PER_WORKER // CHUNK

_mesh = plsc.VectorSubcoreMesh(core_axis_name="c", subcore_axis_name="s")


@functools.partial(
    pl.kernel,
    mesh=_mesh,
    out_type=jax.ShapeDtypeStruct((SC_ROWS, DIM), jnp.float32),
    scratch_types=[
        pltpu.VMEM((CHUNK, DIM), jnp.float32),
        pltpu.VMEM((CHUNK, DIM), jnp.float32),
        pltpu.SemaphoreType.DMA,
        pltpu.SemaphoreType.DMA,
        pltpu.SemaphoreType.DMA,
        pltpu.SemaphoreType.DMA,
    ],
)
def _sc_lookup(table_hbm, out_hbm, buf0, buf1, gs0, gs1, ss0, ss1):
    wid = lax.axis_index("s") * NUM_CORES + lax.axis_index("c")
    base = wid * ROWS_PER_WORKER
    bufs = (buf0, buf1)
    gsems = (gs0, gs1)
    ssems = (ss0, ss1)

    def fire_g(i):
        b = i & 1
        return pltpu.async_copy(
            table_hbm.at[pl.ds(base + i * CHUNK, CHUNK)], bufs[b], gsems[b]
        )

    def fire_s(i):
        b = i & 1
        return pltpu.async_copy(
            bufs[b], out_hbm.at[pl.ds(base + i * CHUNK, CHUNK)], ssems[b]
        )

    g0 = fire_g(0)
    g1 = fire_g(1)
    g0.wait()
    s0 = fire_s(0)
    g1.wait()
    s1 = fire_s(1)
    s0.wait()
    s1.wait()


TC_BLOCK = 2048
TC_GRID = SEQ_LEN // TC_BLOCK


def _copy_body(in_ref, out_ref):
    out_ref[...] = in_ref[...]


def _tc_copy(table):
    return pl.pallas_call(
        _copy_body,
        grid=(TC_GRID,),
        in_specs=[pl.BlockSpec((TC_BLOCK, DIM), lambda i: (i, 0))],
        out_specs=pl.BlockSpec((TC_BLOCK, DIM), lambda i: (i, 0)),
        out_shape=jax.ShapeDtypeStruct((SEQ_LEN, DIM), jnp.float32),
    )(table)


def kernel(x, emb_weight):
    del x
    sc_part = _sc_lookup(emb_weight)
    tc_full = _tc_copy(emb_weight)
    return lax.dynamic_update_slice(tc_full, sc_part, (0, 0))


# speedup vs baseline: 1.0463x; 1.0463x over previous
import functools

import jax
import jax.numpy as jnp
from jax import lax
from jax.experimental import pallas as pl
from jax.experimental.pallas import tpu as pltpu
from jax.experimental.pallas import tpu_sc as plsc

SEQ_LEN = 8192
DIM = 1024
NUM_CORES = 2
NUM_SUBCORES = 16
NUM_WORKERS = NUM_CORES * NUM_SUBCORES
ROWS_PER_WORKER = SEQ_LEN // NUM_WORKERS

NBUF = 3
CHUNK = 40
_SIZES = [CHUNK] * 6 + [ROWS_PER_WORKER - 6 * CHUNK]
_OFFS = [sum(_SIZES[:i]) for i in range(len(_SIZES))]
NCH = len(_SIZES)

_mesh = plsc.VectorSubcoreMesh(core_axis_name="c", subcore_axis_name="s")


@functools.partial(
    pl.kernel,
    mesh=_mesh,
    out_type=jax.ShapeDtypeStruct((SEQ_LEN, DIM), jnp.float32),
    scratch_types=[
        pltpu.VMEM((CHUNK, DIM), jnp.float32),
        pltpu.VMEM((CHUNK, DIM), jnp.float32),
        pltpu.VMEM((CHUNK, DIM), jnp.float32),
        pltpu.SemaphoreType.DMA,
        pltpu.SemaphoreType.DMA,
        pltpu.SemaphoreType.DMA,
        pltpu.SemaphoreType.DMA,
        pltpu.SemaphoreType.DMA,
        pltpu.SemaphoreType.DMA,
    ],
)
def _pos_embed_lookup(
    table_hbm, out_hbm, buf0, buf1, buf2, gs0, gs1, gs2, ss0, ss1, ss2
):
    wid = lax.axis_index("s") * NUM_CORES + lax.axis_index("c")
    base = wid * ROWS_PER_WORKER
    bufs = (buf0, buf1, buf2)
    gsems = (gs0, gs1, gs2)
    ssems = (ss0, ss1, ss2)

    def fire_g(i):
        b = i % NBUF
        return pltpu.async_copy(
            table_hbm.at[pl.ds(base + _OFFS[i], _SIZES[i])],
            bufs[b].at[pl.ds(0, _SIZES[i])],
            gsems[b],
        )

    def fire_s(i):
        b = i % NBUF
        return pltpu.async_copy(
            bufs[b].at[pl.ds(0, _SIZES[i])],
            out_hbm.at[pl.ds(base + _OFFS[i], _SIZES[i])],
            ssems[b],
        )

    g = [None] * NCH
    s = [None] * NCH
    for j in range(NBUF):
        g[j] = fire_g(j)
    for i in range(NCH):
        if i >= NBUF:
            s[i - NBUF].wait()
            g[i] = fire_g(i)
        g[i].wait()
        s[i] = fire_s(i)
    for i in range(NCH - NBUF, NCH):
        s[i].wait()


def kernel(x, emb_weight):
    del x
    return _pos_embed_lookup(emb_weight)


# speedup vs baseline: 1.0646x; 1.0175x over previous
import functools

import jax
import jax.numpy as jnp
from jax import lax
from jax.experimental import pallas as pl
from jax.experimental.pallas import tpu as pltpu
from jax.experimental.pallas import tpu_sc as plsc

SEQ_LEN = 8192
DIM = 1024
NUM_CORES = 2
NUM_SUBCORES = 16
NUM_WORKERS = NUM_CORES * NUM_SUBCORES
ROWS_PER_WORKER = SEQ_LEN // NUM_WORKERS

_SIZES = [64, 56, 64, 56, 16]
_OFFS = [sum(_SIZES[:i]) for i in range(len(_SIZES))]
NCH = len(_SIZES)

_mesh = plsc.VectorSubcoreMesh(core_axis_name="c", subcore_axis_name="s")


@functools.partial(
    pl.kernel,
    mesh=_mesh,
    out_type=jax.ShapeDtypeStruct((SEQ_LEN, DIM), jnp.float32),
    scratch_types=[
        pltpu.VMEM((64, DIM), jnp.float32),
        pltpu.VMEM((56, DIM), jnp.float32),
        pltpu.SemaphoreType.DMA,
        pltpu.SemaphoreType.DMA,
        pltpu.SemaphoreType.DMA,
        pltpu.SemaphoreType.DMA,
    ],
)
def _pos_embed_lookup(table_hbm, out_hbm, buf0, buf1, gs0, gs1, ss0, ss1):
    wid = lax.axis_index("s") * NUM_CORES + lax.axis_index("c")
    base = wid * ROWS_PER_WORKER
    bufs = (buf0, buf1)
    gsems = (gs0, gs1)
    ssems = (ss0, ss1)

    def fire_g(i):
        b = i & 1
        return pltpu.async_copy(
            table_hbm.at[pl.ds(base + _OFFS[i], _SIZES[i])],
            bufs[b].at[pl.ds(0, _SIZES[i])],
            gsems[b],
        )

    def fire_s(i):
        b = i & 1
        return pltpu.async_copy(
            bufs[b].at[pl.ds(0, _SIZES[i])],
            out_hbm.at[pl.ds(base + _OFFS[i], _SIZES[i])],
            ssems[b],
        )

    g = [None] * NCH
    s = [None] * NCH
    g[0] = fire_g(0)
    g[1] = fire_g(1)
    g[0].wait()
    s[0] = fire_s(0)
    g[1].wait()
    s[1] = fire_s(1)
    for i in range(2, NCH):
        s[i - 2].wait()
        g[i] = fire_g(i)
        g[i].wait()
        s[i] = fire_s(i)
    s[NCH - 2].wait()
    s[NCH - 1].wait()


def kernel(x, emb_weight):
    del x
    return _pos_embed_lookup(emb_weight)
